# baseline (speedup 1.0000x reference)
import functools

import jax
import jax.numpy as jnp
from jax.experimental import pallas as pl
from jax.experimental.pallas import tpu as pltpu


def _route(logits, coef):
    b_dim, m_dim = logits.shape
    col = jax.lax.broadcasted_iota(jnp.int32, (b_dim, m_dim), 1)
    big = jnp.int32(m_dim)

    m1 = jnp.max(logits, axis=1, keepdims=True)
    idx1 = jnp.min(jnp.where(logits == m1, col, big), axis=1, keepdims=True)
    masked = jnp.where(col == idx1, -jnp.inf, logits)
    m2 = jnp.max(masked, axis=1, keepdims=True)
    idx2 = jnp.min(jnp.where(masked == m2, col, big), axis=1, keepdims=True)

    e = jnp.exp(m2 - m1)
    denom = 1.0 + e
    g1 = 1.0 / denom
    g2 = e / denom
    gates = (jnp.where(col == idx1, g1, 0.0)
             + jnp.where(col == idx2, g2, 0.0))

    imp = jnp.sum(gates, axis=0, keepdims=True)
    load = jnp.sum((gates > 0.0).astype(jnp.float32), axis=0, keepdims=True)

    def cv_sq(v):
        mean = jnp.sum(v) * jnp.float32(1.0 / m_dim)
        var = jnp.sum((v - mean) ** 2) * jnp.float32(1.0 / (m_dim - 1))
        return var / (mean * mean + jnp.float32(1e-10))

    loss = (cv_sq(imp) + cv_sq(load)) * coef
    return gates, loss


def _fused_body(bb, nsteps, x_ref, w_ref, coef_ref, g_ref,
                loss_ref, wt_buf, logit_buf):
    i = pl.program_id(0)
    s = x_ref.shape[1]
    m = w_ref.shape[1]

    @pl.when(i == 0)
    def _prep():
        wt_buf[...] = w_ref[...].T.astype(jnp.bfloat16).astype(jnp.float32)

    ssum = jnp.sum(x_ref[...], axis=1)
    mean = (ssum * jnp.float32(1.0 / s)).astype(jnp.bfloat16).astype(
        jnp.float32)
    prod = mean[:, None, :] * wt_buf[...][None, :, :]
    blk = jnp.sum(prod, axis=2)

    @pl.when(i < nsteps - 1)
    def _store():
        logit_buf[i, :, :] = blk

    @pl.when(i == nsteps - 1)
    def _epilogue():
        head = logit_buf[0:nsteps - 1, :, :]
        lg = jnp.concatenate(
            [head.reshape((nsteps - 1) * bb, m), blk], axis=0)
        gates, loss = _route(lg, coef_ref[0])
        g_ref[...] = gates
        loss_ref[0, 0] = loss


def kernel(feats, w_gate, w_noise, loss_coef=0.01, noise_epsilon=0.01):
    B, C, H, W = feats.shape
    S = H * W
    M = w_gate.shape[1]
    x = jnp.transpose(feats, (0, 2, 3, 1)).reshape(B, S, C)
    BB = 2
    coef = jnp.reshape(jnp.asarray(loss_coef, jnp.float32), (1,))

    NSTEPS = B // BB
    gates, loss = pl.pallas_call(
        functools.partial(_fused_body, BB, NSTEPS),
        grid=(NSTEPS,),
        in_specs=[
            pl.BlockSpec((BB, S, C), lambda i: (i, 0, 0)),
            pl.BlockSpec((C, M), lambda i: (0, 0)),
            pl.BlockSpec(memory_space=pltpu.SMEM),
        ],
        out_specs=[
            pl.BlockSpec((B, M), lambda i: (0, 0)),
            pl.BlockSpec(memory_space=pltpu.SMEM),
        ],
        out_shape=[
            jax.ShapeDtypeStruct((B, M), jnp.float32),
            jax.ShapeDtypeStruct((1, 1), jnp.float32),
        ],
        scratch_shapes=[
            pltpu.VMEM((M, C), jnp.float32),
            pltpu.VMEM((NSTEPS, BB, M), jnp.float32),
        ],
    )(x, w_gate, coef)

    return gates, loss[0, 0]


# speedup vs baseline: 1.1472x; 1.1472x over previous
import functools

import jax
import jax.numpy as jnp
from jax.experimental import pallas as pl
from jax.experimental.pallas import tpu as pltpu


def _route(logits, coef):
    b_dim, m_dim = logits.shape
    col = jax.lax.broadcasted_iota(jnp.int32, (b_dim, m_dim), 1)
    big = jnp.int32(m_dim)

    m1 = jnp.max(logits, axis=1, keepdims=True)
    idx1 = jnp.min(jnp.where(logits == m1, col, big), axis=1, keepdims=True)
    masked = jnp.where(col == idx1, -jnp.inf, logits)
    m2 = jnp.max(masked, axis=1, keepdims=True)
    idx2 = jnp.min(jnp.where(masked == m2, col, big), axis=1, keepdims=True)

    e = jnp.exp(m2 - m1)
    denom = 1.0 + e
    g1 = 1.0 / denom
    g2 = e / denom
    gates = (jnp.where(col == idx1, g1, 0.0)
             + jnp.where(col == idx2, g2, 0.0))

    imp = jnp.sum(gates, axis=0, keepdims=True)
    load = jnp.sum((gates > 0.0).astype(jnp.float32), axis=0, keepdims=True)

    def cv_sq(v):
        mean = jnp.sum(v) * jnp.float32(1.0 / m_dim)
        var = jnp.sum((v - mean) ** 2) * jnp.float32(1.0 / (m_dim - 1))
        return var / (mean * mean + jnp.float32(1e-10))

    loss = (cv_sq(imp) + cv_sq(load)) * coef
    return gates, loss


def _fused_body(bb, nsteps, x_ref, w_ref, coef_ref, g_ref,
                loss_ref, wt_buf, logit_buf):
    i = pl.program_id(0)
    s = x_ref.shape[1]
    m = w_ref.shape[1]

    @pl.when(i == 0)
    def _prep():
        wt_buf[...] = w_ref[...].T.astype(jnp.bfloat16).astype(jnp.float32)

    ssum = jnp.sum(x_ref[...], axis=1)
    mean = (ssum * jnp.float32(1.0 / s)).astype(jnp.bfloat16).astype(
        jnp.float32)
    prod = mean[:, None, :] * wt_buf[...][None, :, :]
    blk = jnp.sum(prod, axis=2)

    @pl.when(i < nsteps - 1)
    def _store():
        logit_buf[i, :, :] = blk

    @pl.when(i == nsteps - 1)
    def _epilogue():
        head = logit_buf[0:nsteps - 1, :, :]
        lg = jnp.concatenate(
            [head.reshape((nsteps - 1) * bb, m), blk], axis=0)
        gates, loss = _route(lg, coef_ref[0])
        g_ref[...] = gates
        loss_ref[0, 0] = loss


def kernel(feats, w_gate, w_noise, loss_coef=0.01, noise_epsilon=0.01):
    B, C, H, W = feats.shape
    S = H * W
    M = w_gate.shape[1]
    x = jnp.transpose(feats, (0, 2, 3, 1)).reshape(B, S, C)
    BB = 4
    coef = jnp.reshape(jnp.asarray(loss_coef, jnp.float32), (1,))

    NSTEPS = B // BB
    gates, loss = pl.pallas_call(
        functools.partial(_fused_body, BB, NSTEPS),
        grid=(NSTEPS,),
        in_specs=[
            pl.BlockSpec((BB, S, C), lambda i: (i, 0, 0)),
            pl.BlockSpec((C, M), lambda i: (0, 0)),
            pl.BlockSpec(memory_space=pltpu.SMEM),
        ],
        out_specs=[
            pl.BlockSpec((B, M), lambda i: (0, 0)),
            pl.BlockSpec(memory_space=pltpu.SMEM),
        ],
        out_shape=[
            jax.ShapeDtypeStruct((B, M), jnp.float32),
            jax.ShapeDtypeStruct((1, 1), jnp.float32),
        ],
        scratch_shapes=[
            pltpu.VMEM((M, C), jnp.float32),
            pltpu.VMEM((NSTEPS, BB, M), jnp.float32),
        ],
    )(x, w_gate, coef)

    return gates, loss[0, 0]
